# baseline (speedup 1.0000x reference)
import jax
import jax.numpy as jnp
from jax.experimental import pallas as pl

_G = 256


def _gcn2_body(atoms_ref, adjs_ref, w1_ref, b1_ref, w2_ref, b2_ref, out_ref):
    g, n, dm = atoms_ref.shape
    a = atoms_ref[...]
    adj = (adjs_ref[...] > 0.5).astype(jnp.float32)
    deg = jnp.sum(adj, axis=1) + 1.0
    d = jax.lax.rsqrt(deg)
    eye = jnp.eye(n, dtype=jnp.float32)
    m = ((adj + eye) * d[:, :, None] * d[:, None, :]).astype(jnp.bfloat16)

    y = jnp.reshape(jnp.reshape(a, (g * n, dm)) @ w1_ref[...], (g, n, dm))
    s = jax.lax.dot_general(m, y.astype(jnp.bfloat16),
                            (((1,), (1,)), ((0,), (0,))),
                            preferred_element_type=jnp.float32)
    h = jnp.maximum(s + b1_ref[...], 0.0)

    y = jnp.reshape(jnp.reshape(h, (g * n, dm)) @ w2_ref[...], (g, n, dm))
    s = jax.lax.dot_general(m, y.astype(jnp.bfloat16),
                            (((1,), (1,)), ((0,), (0,))),
                            preferred_element_type=jnp.float32)
    out_ref[...] = s + b2_ref[...]


def kernel(atoms, adjs, W1, b1, W2, b2):
    batch, n, dm = atoms.shape
    b1r = jnp.reshape(b1, (1, dm))
    b2r = jnp.reshape(b2, (1, dm))
    grid = (batch // _G,)
    return pl.pallas_call(
        _gcn2_body,
        grid=grid,
        in_specs=[
            pl.BlockSpec((_G, n, dm), lambda i: (i, 0, 0)),
            pl.BlockSpec((_G, n, n), lambda i: (i, 0, 0)),
            pl.BlockSpec((dm, dm), lambda i: (0, 0)),
            pl.BlockSpec((1, dm), lambda i: (0, 0)),
            pl.BlockSpec((dm, dm), lambda i: (0, 0)),
            pl.BlockSpec((1, dm), lambda i: (0, 0)),
        ],
        out_specs=pl.BlockSpec((_G, n, dm), lambda i: (i, 0, 0)),
        out_shape=jax.ShapeDtypeStruct((batch, n, dm), jnp.float32),
    )(atoms, adjs, W1, b1r, W2, b2r)


# speedup vs baseline: 1.1396x; 1.1396x over previous
import jax
import jax.numpy as jnp
from jax.experimental import pallas as pl

_G = 256


def _gcn2_body(atoms_ref, adjs_ref, w1_ref, b1_ref, w2_ref, b2_ref, out_ref):
    g, n, dm = atoms_ref.shape
    a = atoms_ref[...]
    adj = (adjs_ref[...] > 0.5).astype(jnp.float32)
    deg = jnp.sum(adj, axis=1) + 1.0
    d = jax.lax.rsqrt(deg)
    eye = jnp.eye(n, dtype=jnp.float32)
    m = (adj + eye) * d[:, :, None] * d[:, None, :]

    y = jnp.reshape(jnp.reshape(a, (g * n, dm)) @ w1_ref[...], (g, n, dm))
    s = jax.lax.dot_general(m, y, (((1,), (1,)), ((0,), (0,))),
                            preferred_element_type=jnp.float32)
    h = jnp.maximum(s + b1_ref[...], 0.0)

    y = jnp.reshape(jnp.reshape(h, (g * n, dm)) @ w2_ref[...], (g, n, dm))
    s = jax.lax.dot_general(m, y, (((1,), (1,)), ((0,), (0,))),
                            preferred_element_type=jnp.float32)
    out_ref[...] = s + b2_ref[...]


def kernel(atoms, adjs, W1, b1, W2, b2):
    batch, n, dm = atoms.shape
    b1r = jnp.reshape(b1, (1, dm))
    b2r = jnp.reshape(b2, (1, dm))
    grid = (batch // _G,)
    return pl.pallas_call(
        _gcn2_body,
        grid=grid,
        in_specs=[
            pl.BlockSpec((_G, n, dm), lambda i: (i, 0, 0)),
            pl.BlockSpec((_G, n, n), lambda i: (i, 0, 0)),
            pl.BlockSpec((dm, dm), lambda i: (0, 0)),
            pl.BlockSpec((1, dm), lambda i: (0, 0)),
            pl.BlockSpec((dm, dm), lambda i: (0, 0)),
            pl.BlockSpec((1, dm), lambda i: (0, 0)),
        ],
        out_specs=pl.BlockSpec((_G, n, dm), lambda i: (i, 0, 0)),
        out_shape=jax.ShapeDtypeStruct((batch, n, dm), jnp.float32),
    )(atoms, adjs, W1, b1r, W2, b2r)
